# Initial kernel scaffold; baseline (speedup 1.0000x reference)
import jax
import jax.numpy as jnp
from jax.experimental import pallas as pl


def kernel(scores, train_ensemble, gumbel):
    raise NotImplementedError("write your pallas kernel here")


# speedup vs baseline: 1.2978x; 1.2978x over previous
import jax
import jax.numpy as jnp
import numpy as np
from jax.experimental import pallas as pl

_EPS = float(np.finfo(np.float32).tiny)
_K = 64
_TAU = 0.1
_BISECT_STEPS = 31
_ROWS_PER_BLOCK = 8


def _gumbel_topk_block(s_ref, g_ref, o_ref):
    fs = s_ref[...] + g_ref[...]
    zeros = jnp.zeros_like(fs)

    def iteration(_, carry):
        fs, khot, onehot = carry
        fs = fs + jnp.log(jnp.maximum(1.0 - onehot, _EPS))
        x = fs / _TAU
        m = jnp.max(x, axis=1, keepdims=True)
        e = jnp.exp(x - m)
        s = jnp.sum(e, axis=1, keepdims=True)
        onehot = e / s
        khot = khot + onehot
        return fs, khot, onehot

    _, khot, _ = jax.lax.fori_loop(0, _K, iteration, (fs, zeros, zeros))

    rows = khot.shape[0]
    lo = jnp.zeros((rows, 1), jnp.int32)
    hi = jnp.full((rows, 1), 0x43000000, jnp.int32)

    def bisect(_, carry):
        lo, hi = carry
        mid = (lo + hi) // 2
        thr = jax.lax.bitcast_convert_type(mid, jnp.float32)
        cnt = jnp.sum(jnp.where(khot >= thr, 1.0, 0.0), axis=1, keepdims=True)
        ge = cnt >= _K
        return jnp.where(ge, mid, lo), jnp.where(ge, hi, mid)

    lo, hi = jax.lax.fori_loop(0, _BISECT_STEPS, bisect, (lo, hi))
    v64 = jax.lax.bitcast_convert_type(lo, jnp.float32)
    hard = jnp.where(khot >= v64, 1.0, 0.0)
    o_ref[...] = (hard - khot) + khot


def kernel(scores, train_ensemble, gumbel):
    bsz, Nmax, ensemble = scores.shape
    te = gumbel.shape[0] // (bsz * ensemble)
    flat_scores = scores.reshape(bsz * ensemble, Nmax)
    r = _ROWS_PER_BLOCK
    out = pl.pallas_call(
        _gumbel_topk_block,
        grid=(bsz * ensemble * te // r,),
        in_specs=[
            pl.BlockSpec((r, Nmax), lambda i: (i, 0)),
            pl.BlockSpec((r, Nmax), lambda i: (i, 0)),
        ],
        out_specs=pl.BlockSpec((r, Nmax), lambda i: (i, 0)),
        out_shape=jax.ShapeDtypeStruct((te * bsz * ensemble, Nmax), jnp.float32),
    )(flat_scores, gumbel)
    return out.reshape(te, bsz, ensemble, Nmax).transpose(0, 1, 3, 2)


# speedup vs baseline: 1.4439x; 1.1126x over previous
import jax
import jax.numpy as jnp
import numpy as np
from jax.experimental import pallas as pl
from jax.experimental.pallas import tpu as pltpu

_EPS = float(np.finfo(np.float32).tiny)
_K = 64
_TAU = 0.1
_BISECT_STEPS = 31
_ROWS_PER_BLOCK = 8


def _gumbel_topk_block(s_ref, g_ref, o_ref, fs_ref, khot_ref):
    fs_ref[...] = s_ref[...] + g_ref[...]
    khot_ref[...] = jnp.zeros_like(khot_ref[...])

    def iteration(_, carry):
        fs = fs_ref[...]
        x = fs / _TAU
        m = jnp.max(x, axis=1, keepdims=True)
        e = jnp.exp(x - m)
        s = jnp.sum(e, axis=1, keepdims=True)
        onehot = e / s
        khot_ref[...] = khot_ref[...] + onehot
        fs_ref[...] = fs + jnp.log(jnp.maximum(1.0 - onehot, _EPS))
        return carry

    jax.lax.fori_loop(0, _K, iteration, 0)
    khot = khot_ref[...]

    rows = khot.shape[0]
    lo = jnp.zeros((rows, 1), jnp.int32)
    hi = jnp.full((rows, 1), 0x43000000, jnp.int32)

    def bisect(_, carry):
        lo, hi = carry
        mid = (lo + hi) // 2
        thr = jax.lax.bitcast_convert_type(mid, jnp.float32)
        cnt = jnp.sum(jnp.where(khot >= thr, 1.0, 0.0), axis=1, keepdims=True)
        ge = cnt >= _K
        return jnp.where(ge, mid, lo), jnp.where(ge, hi, mid)

    lo, hi = jax.lax.fori_loop(0, _BISECT_STEPS, bisect, (lo, hi))
    v64 = jax.lax.bitcast_convert_type(lo, jnp.float32)
    hard = jnp.where(khot >= v64, 1.0, 0.0)
    o_ref[...] = (hard - khot) + khot


def kernel(scores, train_ensemble, gumbel):
    bsz, Nmax, ensemble = scores.shape
    te = gumbel.shape[0] // (bsz * ensemble)
    flat_scores = scores.reshape(bsz * ensemble, Nmax)
    r = _ROWS_PER_BLOCK
    out = pl.pallas_call(
        _gumbel_topk_block,
        grid=(bsz * ensemble * te // r,),
        in_specs=[
            pl.BlockSpec((r, Nmax), lambda i: (i, 0)),
            pl.BlockSpec((r, Nmax), lambda i: (i, 0)),
        ],
        out_specs=pl.BlockSpec((r, Nmax), lambda i: (i, 0)),
        out_shape=jax.ShapeDtypeStruct((te * bsz * ensemble, Nmax), jnp.float32),
        scratch_shapes=[
            pltpu.VMEM((r, Nmax), jnp.float32),
            pltpu.VMEM((r, Nmax), jnp.float32),
        ],
    )(flat_scores, gumbel)
    return out.reshape(te, bsz, ensemble, Nmax).transpose(0, 1, 3, 2)


# speedup vs baseline: 1.9178x; 1.3282x over previous
import jax
import jax.numpy as jnp
import numpy as np
from jax.experimental import pallas as pl
from jax.experimental.pallas import tpu as pltpu

_EPS = float(np.finfo(np.float32).tiny)
_K = 64
_TAU = 0.1
_BISECT_STEPS = 31
_ROWS_PER_BLOCK = 16


def _gumbel_topk_block(s_ref, g_ref, o_ref, fs_ref, khot_ref):
    fs_ref[...] = s_ref[...] + g_ref[...]
    khot_ref[...] = jnp.zeros_like(khot_ref[...])

    def iteration(_, carry):
        fs = fs_ref[...]
        x = fs / _TAU
        m = jnp.max(x, axis=1, keepdims=True)
        e = jnp.exp(x - m)
        s = jnp.sum(e, axis=1, keepdims=True)
        onehot = e / s
        khot_ref[...] = khot_ref[...] + onehot
        fs_ref[...] = fs + jnp.log(jnp.maximum(1.0 - onehot, _EPS))
        return carry

    jax.lax.fori_loop(0, _K, iteration, 0)
    khot = khot_ref[...]

    rows = khot.shape[0]
    lo = jnp.zeros((rows, 1), jnp.int32)
    hi = jnp.full((rows, 1), 0x43000000, jnp.int32)

    def bisect(_, carry):
        lo, hi = carry
        mid = (lo + hi) // 2
        thr = jax.lax.bitcast_convert_type(mid, jnp.float32)
        cnt = jnp.sum(jnp.where(khot >= thr, 1.0, 0.0), axis=1, keepdims=True)
        ge = cnt >= _K
        return jnp.where(ge, mid, lo), jnp.where(ge, hi, mid)

    lo, hi = jax.lax.fori_loop(0, _BISECT_STEPS, bisect, (lo, hi))
    v64 = jax.lax.bitcast_convert_type(lo, jnp.float32)
    hard = jnp.where(khot >= v64, 1.0, 0.0)
    o_ref[...] = (hard - khot) + khot


def kernel(scores, train_ensemble, gumbel):
    bsz, Nmax, ensemble = scores.shape
    te = gumbel.shape[0] // (bsz * ensemble)
    flat_scores = scores.reshape(bsz * ensemble, Nmax)
    r = _ROWS_PER_BLOCK
    out = pl.pallas_call(
        _gumbel_topk_block,
        grid=(bsz * ensemble * te // r,),
        in_specs=[
            pl.BlockSpec((r, Nmax), lambda i: (i, 0)),
            pl.BlockSpec((r, Nmax), lambda i: (i, 0)),
        ],
        out_specs=pl.BlockSpec((r, Nmax), lambda i: (i, 0)),
        out_shape=jax.ShapeDtypeStruct((te * bsz * ensemble, Nmax), jnp.float32),
        scratch_shapes=[
            pltpu.VMEM((r, Nmax), jnp.float32),
            pltpu.VMEM((r, Nmax), jnp.float32),
        ],
    )(flat_scores, gumbel)
    return out.reshape(te, bsz, ensemble, Nmax).transpose(0, 1, 3, 2)
